# baseline (speedup 1.0000x reference)
import functools

import jax
import jax.numpy as jnp
import numpy as np
from jax import lax
from jax.experimental import pallas as pl
from jax.experimental.pallas import tpu as pltpu
from jax.experimental.pallas import tpu_sc as plsc

_LOG2E = 1.4426950408889634


def _dense_body(x_ref, w_ref, o_ref, q_ref):
    x = x_ref[...]
    u = lax.exp2(jnp.abs(x) * (-_LOG2E))
    sp = jnp.maximum(x, 0.0) + jnp.log(1.0 + u)
    o_ref[...] = jnp.full((1, 1, 128), jnp.sum(sp * w_ref[...][0]), jnp.float32)
    bk, s = x.shape[0], x.shape[1]
    r = bk * s
    x8 = x[:, :, :, :8]
    x8p = jnp.concatenate([x8, jnp.zeros((bk, s, 12, 8), jnp.float32)], axis=2)
    for j in range(4):
        c = x8p[:, :, 16 * j:16 * j + 16, :].reshape(r, 16, 8)
        q_ref[j] = c.reshape(r, 128)


def _dense_sum(p, wts, grid):
    b, s, _, ch = p.shape
    bk = b // grid
    out, q = pl.pallas_call(
        _dense_body,
        grid=(grid,),
        in_specs=[
            pl.BlockSpec((bk, s, s, ch), lambda i: (i, 0, 0, 0)),
            pl.BlockSpec((1, ch), lambda i: (0, 0)),
        ],
        out_specs=[
            pl.BlockSpec((1, 1, 128), lambda i: (i, 0, 0)),
            pl.BlockSpec((4, bk * s, 128), lambda i: (0, i, 0)),
        ],
        out_shape=[
            jax.ShapeDtypeStruct((grid, 1, 128), jnp.float32),
            jax.ShapeDtypeStruct((4, b * s, 128), jnp.float32),
        ],
    )(p, wts)
    return jnp.sum(out[:, 0, 0]), q


def _make_sparse(B, T, S, CH):
    info = plsc.get_sparse_core_info()
    NC, NS, L = info.num_cores, info.num_subcores, info.num_lanes
    NW = NC * NS
    SB = B // NW
    NCH = (T + L - 1) // L
    TP = NCH * L
    CELLS = S * S
    WBUF = 1 << (CELLS + TP - 1).bit_length()
    TROW = 512
    QR = B * S

    mesh = plsc.VectorSubcoreMesh(core_axis_name="c", subcore_axis_name="s")

    @functools.partial(
        pl.kernel,
        mesh=mesh,
        compiler_params=pltpu.CompilerParams(needs_layout_passes=False),
        out_type=jax.ShapeDtypeStruct((NW, 4, L), jnp.float32),
        scratch_types=(
            [pltpu.VMEM((TROW,), jnp.float32) for _ in range(SB)]
            + [pltpu.VMEM((TP, 128), jnp.float32) for _ in range(SB)]
            + [
                pltpu.VMEM((WBUF,), jnp.int32),
                pltpu.VMEM((SB, TP), jnp.int32),
                pltpu.VMEM((SB, TP), jnp.int32),
                pltpu.VMEM((SB, TP), jnp.int32),
                pltpu.VMEM((SB, TP), jnp.int32),
                pltpu.VMEM((SB, TP), jnp.float32),
                pltpu.VMEM((4, L), jnp.float32),
                pltpu.SemaphoreType.DMA,
                pltpu.SemaphoreType.DMA,
            ]
        ),
    )
    def sparse(q_hbm, t_hbm, out_hbm, *scr):
        tvs = scr[:SB]
        vals = scr[SB:2 * SB]
        (winbuf, cellsv, clsv, idxv, colv, maskv, accv, sem_t, sem_q) = scr[2 * SB:]
        cid = lax.axis_index("c")
        sid = lax.axis_index("s")
        wid = sid * NC + cid
        iota = lax.iota(jnp.int32, L)
        sf = jnp.float32(S)

        tcopies = [
            pltpu.async_copy(t_hbm.at[wid * SB + k], tvs[k], sem_t)
            for k in range(SB)
        ]
        for cp in tcopies:
            cp.wait()

        qcopies = []
        for k in range(SB):
            b = wid * SB + k
            for u in range(NCH):
                j16 = u * L + iota
                valid = j16 < T
                js = jnp.minimum(j16, T - 1)
                x = plsc.load_gather(tvs[k], [5 * js + 1])
                y = plsc.load_gather(tvs[k], [5 * js + 2])
                cf = plsc.load_gather(tvs[k], [5 * js])
                gx = (x * sf).astype(jnp.int32)
                gy = (y * sf).astype(jnp.int32)
                cell = gy * S + gx
                cell = jnp.where(valid, cell, CELLS + j16)
                cellsv[k, pl.ds(u * L, L)] = cell
                clsv[k, pl.ds(u * L, L)] = cf.astype(jnp.int32)
                rowid = lax.shift_right_logical(gx, 4) * QR + b * S + gy
                idxv[k, pl.ds(u * L, L)] = jnp.where(valid, rowid, 0)
                colv[k, pl.ds(u * L, L)] = (gx & 15) * 8
                plsc.store_scatter(winbuf, [cell], j16, mask=valid)
            qcopies.append(pltpu.async_copy(q_hbm.at[idxv.at[k]], vals[k], sem_q))
            for u in range(NCH):
                j16 = u * L + iota
                valid = j16 < T
                cell = cellsv[k, pl.ds(u * L, L)]
                wv = plsc.load_gather(winbuf, [cell])
                is_win = (wv == j16) & valid
                maskv[k, pl.ds(u * L, L)] = jnp.where(is_win, 1.0, 0.0)

        for cp in qcopies:
            cp.wait()

        acc_mse = jnp.zeros((L,), jnp.float32)
        acc_p4 = jnp.zeros((L,), jnp.float32)
        acc_pc = jnp.zeros((L,), jnp.float32)
        for k in range(SB):
            for u in range(NCH):
                j16 = u * L + iota
                js = jnp.minimum(j16, T - 1)
                m = maskv[k, pl.ds(u * L, L)]
                tx = plsc.load_gather(tvs[k], [5 * js + 1])
                ty = plsc.load_gather(tvs[k], [5 * js + 2])
                tw = plsc.load_gather(tvs[k], [5 * js + 3])
                th = plsc.load_gather(tvs[k], [5 * js + 4])
                cc = clsv[k, pl.ds(u * L, L)]
                cb = colv[k, pl.ds(u * L, L)]
                r16 = u * L + iota
                p0 = plsc.load_gather(vals[k], [r16, cb])
                p1 = plsc.load_gather(vals[k], [r16, cb + 1])
                p2 = plsc.load_gather(vals[k], [r16, cb + 2])
                p3 = plsc.load_gather(vals[k], [r16, cb + 3])
                p4 = plsc.load_gather(vals[k], [r16, cb + 4])
                p5 = plsc.load_gather(vals[k], [r16, cb + 5 + cc])
                d0 = p0 - tx
                d1 = p1 - ty
                d2 = p2 - tw
                d3 = p3 - th
                acc_mse = acc_mse + m * (d0 * d0 + d1 * d1 + d2 * d2 + d3 * d3)
                acc_p4 = acc_p4 + m * p4
                acc_pc = acc_pc + m * p5

        accv[0, :] = acc_mse
        accv[1, :] = acc_p4
        accv[2, :] = acc_pc
        accv[3, :] = jnp.zeros((L,), jnp.float32)
        pltpu.sync_copy(accv, out_hbm.at[wid])

    return sparse


def kernel(p, targets, S):
    B = p.shape[0]
    T = targets.shape[1]
    CH = p.shape[-1]
    C = CH - 5
    Ss = p.shape[1]
    N1 = B * Ss * Ss
    N2 = N1 * C

    t2 = jnp.pad(targets.reshape(B, T * 5), ((0, 0), (0, 512 - T * 5)))

    wts = np.zeros((1, CH), np.float32)
    wts[0, 4] = float(C)
    wts[0, 5:] = 1.0
    dense, q = _dense_sum(p, jnp.asarray(wts), 64)
    q2 = q.reshape(4 * B * Ss, 128)

    sc = _make_sparse(B, T, Ss, CH)(q2, t2)
    s_mse = jnp.sum(sc[:, 0, :])
    s_p4 = jnp.sum(sc[:, 1, :])
    s_pc = jnp.sum(sc[:, 2, :])

    return (dense / N2 + s_mse / (4.0 * N1) - s_p4 / N1 - s_pc / N2).astype(p.dtype)


# speedup vs baseline: 2.5544x; 2.5544x over previous
import functools

import jax
import jax.numpy as jnp
import numpy as np
from jax import lax
from jax.experimental import pallas as pl
from jax.experimental.pallas import tpu as pltpu
from jax.experimental.pallas import tpu_sc as plsc

_LOG2E = 1.4426950408889634


def _dense_body(x_ref, w_ref, o_ref, q_ref):
    x = x_ref[...]
    u = lax.exp2(jnp.abs(x) * (-_LOG2E))
    sp = jnp.maximum(x, 0.0) + jnp.log(1.0 + u)
    o_ref[...] = jnp.full((1, 1, 128), jnp.sum(sp * w_ref[...]), jnp.float32)
    yb, s = x.shape[0], x.shape[1]
    x8 = x[:, :, :8, :]
    x8p = jnp.concatenate([x8, jnp.zeros((yb, 12, 8, 128), jnp.float32)],
                          axis=1)
    for y in range(yb):
        for j in range(4):
            m = x8p[y, 16 * j:16 * j + 16].reshape(128, 128)
            q_ref[j, pl.ds(y * 128, 128)] = m.T


def _dense_sum(pt, wts, grid):
    s, _, ch, b = pt.shape
    yb = s // grid
    out, q = pl.pallas_call(
        _dense_body,
        grid=(grid,),
        in_specs=[
            pl.BlockSpec((yb, s, ch, b), lambda i: (i, 0, 0, 0)),
            pl.BlockSpec((ch, b), lambda i: (0, 0)),
        ],
        out_specs=[
            pl.BlockSpec((1, 1, 128), lambda i: (i, 0, 0)),
            pl.BlockSpec((4, yb * 128, 128), lambda i: (0, i, 0)),
        ],
        out_shape=[
            jax.ShapeDtypeStruct((grid, 1, 128), jnp.float32),
            jax.ShapeDtypeStruct((4, s * 128, 128), jnp.float32),
        ],
    )(pt, wts)
    return jnp.sum(out[:, 0, 0]), q


def _make_sparse(B, T, S, CH):
    info = plsc.get_sparse_core_info()
    NC, NS, L = info.num_cores, info.num_subcores, info.num_lanes
    NW = NC * NS
    SB = B // NW
    NCH = (T + L - 1) // L
    TP = NCH * L
    CELLS = S * S
    WBUF = 1 << (CELLS + TP - 1).bit_length()
    TROW = 512
    QR = B * S

    mesh = plsc.VectorSubcoreMesh(core_axis_name="c", subcore_axis_name="s")

    @functools.partial(
        pl.kernel,
        mesh=mesh,
        compiler_params=pltpu.CompilerParams(needs_layout_passes=False),
        out_type=jax.ShapeDtypeStruct((NW, 4, L), jnp.float32),
        scratch_types=(
            [pltpu.VMEM((TROW,), jnp.float32) for _ in range(SB)]
            + [pltpu.VMEM((TP, 128), jnp.float32) for _ in range(SB)]
            + [
                pltpu.VMEM((WBUF,), jnp.int32),
                pltpu.VMEM((SB, TP), jnp.int32),
                pltpu.VMEM((SB, TP), jnp.int32),
                pltpu.VMEM((SB, TP), jnp.int32),
                pltpu.VMEM((SB, TP), jnp.int32),
                pltpu.VMEM((SB, TP), jnp.float32),
                pltpu.VMEM((4, L), jnp.float32),
                pltpu.SemaphoreType.DMA,
                pltpu.SemaphoreType.DMA,
            ]
        ),
    )
    def sparse(q_hbm, t_hbm, out_hbm, *scr):
        tvs = scr[:SB]
        vals = scr[SB:2 * SB]
        (winbuf, cellsv, clsv, idxv, colv, maskv, accv, sem_t, sem_q) = scr[2 * SB:]
        cid = lax.axis_index("c")
        sid = lax.axis_index("s")
        wid = sid * NC + cid
        iota = lax.iota(jnp.int32, L)
        sf = jnp.float32(S)

        tcopies = [
            pltpu.async_copy(t_hbm.at[wid * SB + k], tvs[k], sem_t)
            for k in range(SB)
        ]
        for cp in tcopies:
            cp.wait()

        qcopies = []
        for k in range(SB):
            b = wid * SB + k
            for u in range(NCH):
                j16 = u * L + iota
                valid = j16 < T
                js = jnp.minimum(j16, T - 1)
                x = plsc.load_gather(tvs[k], [5 * js + 1])
                y = plsc.load_gather(tvs[k], [5 * js + 2])
                cf = plsc.load_gather(tvs[k], [5 * js])
                gx = (x * sf).astype(jnp.int32)
                gy = (y * sf).astype(jnp.int32)
                cell = gy * S + gx
                cell = jnp.where(valid, cell, CELLS + j16)
                cellsv[k, pl.ds(u * L, L)] = cell
                clsv[k, pl.ds(u * L, L)] = cf.astype(jnp.int32)
                rowid = lax.shift_right_logical(gx, 4) * QR + gy * B + b
                idxv[k, pl.ds(u * L, L)] = jnp.where(valid, rowid, 0)
                colv[k, pl.ds(u * L, L)] = (gx & 15) * 8
                plsc.store_scatter(winbuf, [cell], j16, mask=valid)
            qcopies.append(pltpu.async_copy(q_hbm.at[idxv.at[k]], vals[k], sem_q))
            for u in range(NCH):
                j16 = u * L + iota
                valid = j16 < T
                cell = cellsv[k, pl.ds(u * L, L)]
                wv = plsc.load_gather(winbuf, [cell])
                is_win = (wv == j16) & valid
                maskv[k, pl.ds(u * L, L)] = jnp.where(is_win, 1.0, 0.0)

        for cp in qcopies:
            cp.wait()

        acc_mse = jnp.zeros((L,), jnp.float32)
        acc_p4 = jnp.zeros((L,), jnp.float32)
        acc_pc = jnp.zeros((L,), jnp.float32)
        for k in range(SB):
            for u in range(NCH):
                j16 = u * L + iota
                js = jnp.minimum(j16, T - 1)
                m = maskv[k, pl.ds(u * L, L)]
                tx = plsc.load_gather(tvs[k], [5 * js + 1])
                ty = plsc.load_gather(tvs[k], [5 * js + 2])
                tw = plsc.load_gather(tvs[k], [5 * js + 3])
                th = plsc.load_gather(tvs[k], [5 * js + 4])
                cc = clsv[k, pl.ds(u * L, L)]
                cb = colv[k, pl.ds(u * L, L)]
                r16 = u * L + iota
                p0 = plsc.load_gather(vals[k], [r16, cb])
                p1 = plsc.load_gather(vals[k], [r16, cb + 1])
                p2 = plsc.load_gather(vals[k], [r16, cb + 2])
                p3 = plsc.load_gather(vals[k], [r16, cb + 3])
                p4 = plsc.load_gather(vals[k], [r16, cb + 4])
                p5 = plsc.load_gather(vals[k], [r16, cb + 5 + cc])
                d0 = p0 - tx
                d1 = p1 - ty
                d2 = p2 - tw
                d3 = p3 - th
                acc_mse = acc_mse + m * (d0 * d0 + d1 * d1 + d2 * d2 + d3 * d3)
                acc_p4 = acc_p4 + m * p4
                acc_pc = acc_pc + m * p5

        accv[0, :] = acc_mse
        accv[1, :] = acc_p4
        accv[2, :] = acc_pc
        accv[3, :] = jnp.zeros((L,), jnp.float32)
        pltpu.sync_copy(accv, out_hbm.at[wid])

    return sparse


def kernel(p, targets, S):
    B = p.shape[0]
    T = targets.shape[1]
    CH = p.shape[-1]
    C = CH - 5
    Ss = p.shape[1]
    N1 = B * Ss * Ss
    N2 = N1 * C

    t2 = jnp.pad(targets.reshape(B, T * 5), ((0, 0), (0, 512 - T * 5)))

    pt = jnp.transpose(p, (1, 2, 3, 0))

    wts = np.zeros((CH, B), np.float32)
    wts[4, :] = float(C)
    wts[5:, :] = 1.0
    dense, q = _dense_sum(pt, jnp.asarray(wts), 26)
    q2 = q.reshape(4 * B * Ss, 128)

    sc = _make_sparse(B, T, Ss, CH)(q2, t2)
    s_mse = jnp.sum(sc[:, 0, :])
    s_p4 = jnp.sum(sc[:, 1, :])
    s_pc = jnp.sum(sc[:, 2, :])

    return (dense / N2 + s_mse / (4.0 * N1) - s_p4 / N1 - s_pc / N2).astype(p.dtype)
